# baseline (speedup 1.0000x reference)
import functools

import jax
import jax.numpy as jnp
from jax import lax
from jax.experimental import pallas as pl
from jax.experimental.pallas import tpu as pltpu
from jax.experimental.pallas import tpu_sc as plsc

D = 64
L = 16
CHUNK = 128


def _fm_sc(upidx, uoff, ipidx, ioff, upair, ipair, params):
    B = upidx.shape[0]
    info = plsc.get_sparse_core_info()
    NC, NS = info.num_cores, info.num_subcores
    NW = NC * NS
    b_per_w = B // NW
    n_chunks = b_per_w // CHUNK

    mesh = plsc.VectorSubcoreMesh(core_axis_name="c", subcore_axis_name="s")

    @functools.partial(
        pl.kernel,
        mesh=mesh,
        out_type=jax.ShapeDtypeStruct((B,), jnp.float32),
        compiler_params=pltpu.CompilerParams(needs_layout_passes=False),
        scratch_types=[
            pltpu.VMEM((b_per_w,), jnp.int32),
            pltpu.VMEM((b_per_w,), jnp.int32),
            pltpu.VMEM((b_per_w,), jnp.int32),
            pltpu.VMEM((b_per_w,), jnp.int32),
            pltpu.VMEM((CHUNK, 2 * D), jnp.float32),
            pltpu.VMEM((CHUNK, 2 * D), jnp.float32),
            pltpu.VMEM((CHUNK, 2 * D), jnp.float32),
            pltpu.VMEM((CHUNK, 2 * D), jnp.float32),
            pltpu.VMEM((D * L + L,), jnp.float32),
            pltpu.VMEM((b_per_w,), jnp.float32),
            pltpu.SemaphoreType.DMA,
            pltpu.SemaphoreType.DMA,
            pltpu.SemaphoreType.DMA,
            pltpu.SemaphoreType.DMA,
        ],
    )
    def k(upidx_hbm, uoff_hbm, ipidx_hbm, ioff_hbm, ut_hbm, it_hbm, p_hbm,
          out_hbm, upidx_v, uoff_v, ipidx_v, ioff_v,
          ub0, ub1, ib0, ib1, w_v, out_v, semu0, semu1, semi0, semi1):
        wid = lax.axis_index("s") * NC + lax.axis_index("c")
        base = wid * b_per_w
        pltpu.sync_copy(upidx_hbm.at[pl.ds(base, b_per_w)], upidx_v)
        pltpu.sync_copy(uoff_hbm.at[pl.ds(base, b_per_w)], uoff_v)
        pltpu.sync_copy(ipidx_hbm.at[pl.ds(base, b_per_w)], ipidx_v)
        pltpu.sync_copy(ioff_hbm.at[pl.ds(base, b_per_w)], ioff_v)
        pltpu.sync_copy(p_hbm, w_v)

        ubufs, ibufs = (ub0, ub1), (ib0, ib1)
        usems, isems = (semu0, semu1), (semi0, semi1)

        def fire(j):
            s = j % 2
            cu = pltpu.async_copy(
                ut_hbm.at[upidx_v.at[pl.ds(j * CHUNK, CHUNK)]], ubufs[s],
                usems[s])
            ci = pltpu.async_copy(
                it_hbm.at[ipidx_v.at[pl.ds(j * CHUNK, CHUNK)]], ibufs[s],
                isems[s])
            return cu, ci

        bias = w_v[pl.ds(D * L, L)]
        lanes = lax.iota(jnp.int32, L)

        pending = fire(0)
        for j in range(n_chunks):
            nxt = fire(j + 1) if j + 1 < n_chunks else None
            for c in pending:
                c.wait()
            pending = nxt
            s = j % 2
            ubuf, ibuf = ubufs[s], ibufs[s]

            def group(g, carry, _j=j, _ubuf=ubuf, _ibuf=ibuf):
                e0 = _j * CHUNK + g * L
                rows = g * L + lanes
                hu = uoff_v[pl.ds(e0, L)]
                hi = ioff_v[pl.ds(e0, L)]
                acc = jnp.zeros((L,), jnp.float32)
                for d in range(D):
                    wv = w_v[pl.ds(d * L, L)]
                    uu = plsc.load_gather(_ubuf, [rows, hu + d])
                    ii = plsc.load_gather(_ibuf, [rows, hi + d])
                    acc = acc + uu * (ii + wv) + ii * wv
                out_v[pl.ds(e0, L)] = acc + bias
                return carry

            lax.fori_loop(0, CHUNK // L, group, 0)

        pltpu.sync_copy(out_v, out_hbm.at[pl.ds(base, b_per_w)])

    return k(upidx, uoff, ipidx, ioff, upair, ipair, params)


def kernel(user_idx, item_idx, user_table, item_table, w, b):
    ui = user_idx.astype(jnp.int32)
    ii = item_idx.astype(jnp.int32)
    upair = user_table.reshape(-1, 2 * D)
    ipair = item_table.reshape(-1, 2 * D)
    params = jnp.concatenate(
        [jnp.repeat(w.astype(jnp.float32), L),
         jnp.broadcast_to(b.astype(jnp.float32), (L,))])
    return _fm_sc(ui >> 1, (ui & 1) << 6, ii >> 1, (ii & 1) << 6,
                  upair, ipair, params)


# speedup vs baseline: 1.5420x; 1.5420x over previous
import functools

import jax
import jax.numpy as jnp
from jax import lax
from jax.experimental import pallas as pl
from jax.experimental.pallas import tpu as pltpu
from jax.experimental.pallas import tpu_sc as plsc

D = 64
L = 16
CHUNK = 128


def _fm_sc(upidx, uoff, ipidx, ioff, upair, ipair, params):
    B = upidx.shape[0]
    info = plsc.get_sparse_core_info()
    NC, NS = info.num_cores, info.num_subcores
    NW = NC * NS
    b_per_w = B // NW
    n_chunks = b_per_w // CHUNK

    mesh = plsc.VectorSubcoreMesh(core_axis_name="c", subcore_axis_name="s")

    @functools.partial(
        pl.kernel,
        mesh=mesh,
        out_type=jax.ShapeDtypeStruct((B,), jnp.float32),
        compiler_params=pltpu.CompilerParams(
            needs_layout_passes=False, use_tc_tiling_on_sc=True),
        scratch_types=[
            pltpu.VMEM((b_per_w,), jnp.int32),
            pltpu.VMEM((b_per_w,), jnp.int32),
            pltpu.VMEM((b_per_w,), jnp.int32),
            pltpu.VMEM((b_per_w,), jnp.int32),
            pltpu.VMEM((CHUNK, 2 * D), jnp.float32),
            pltpu.VMEM((CHUNK, 2 * D), jnp.float32),
            pltpu.VMEM((CHUNK, 2 * D), jnp.float32),
            pltpu.VMEM((CHUNK, 2 * D), jnp.float32),
            pltpu.VMEM((D * L + L,), jnp.float32),
            pltpu.VMEM((b_per_w,), jnp.float32),
            pltpu.SemaphoreType.DMA,
            pltpu.SemaphoreType.DMA,
            pltpu.SemaphoreType.DMA,
            pltpu.SemaphoreType.DMA,
        ],
    )
    def k(upidx_hbm, uoff_hbm, ipidx_hbm, ioff_hbm, ut_hbm, it_hbm, p_hbm,
          out_hbm, upidx_v, uoff_v, ipidx_v, ioff_v,
          ub0, ub1, ib0, ib1, w_v, out_v, semu0, semu1, semi0, semi1):
        wid = lax.axis_index("s") * NC + lax.axis_index("c")
        base = wid * b_per_w
        pltpu.sync_copy(upidx_hbm.at[pl.ds(base, b_per_w)], upidx_v)
        pltpu.sync_copy(uoff_hbm.at[pl.ds(base, b_per_w)], uoff_v)
        pltpu.sync_copy(ipidx_hbm.at[pl.ds(base, b_per_w)], ipidx_v)
        pltpu.sync_copy(ioff_hbm.at[pl.ds(base, b_per_w)], ioff_v)
        pltpu.sync_copy(p_hbm, w_v)

        ubufs, ibufs = (ub0, ub1), (ib0, ib1)
        usems, isems = (semu0, semu1), (semi0, semi1)

        def fire(j):
            s = j % 2
            cu = pltpu.async_copy(
                ut_hbm.at[upidx_v.at[pl.ds(j * CHUNK, CHUNK)]], ubufs[s],
                usems[s])
            ci = pltpu.async_copy(
                it_hbm.at[ipidx_v.at[pl.ds(j * CHUNK, CHUNK)]], ibufs[s],
                isems[s])
            return cu, ci

        bias = w_v[pl.ds(D * L, L)]
        lanes = lax.iota(jnp.int32, L)

        pending = fire(0)
        for j in range(n_chunks):
            nxt = fire(j + 1) if j + 1 < n_chunks else None
            for c in pending:
                c.wait()
            pending = nxt
            s = j % 2
            ubuf, ibuf = ubufs[s], ibufs[s]

            def group(g, carry, _j=j, _ubuf=ubuf, _ibuf=ibuf):
                e0 = _j * CHUNK + g * L
                rows = g * L + lanes
                hu = uoff_v[pl.ds(e0, L)]
                hi = ioff_v[pl.ds(e0, L)]
                acc = jnp.zeros((L,), jnp.float32)
                for d in range(D):
                    wv = w_v[pl.ds(d * L, L)]
                    uu = plsc.load_gather(_ubuf, [rows, hu + d])
                    ii = plsc.load_gather(_ibuf, [rows, hi + d])
                    acc = acc + uu * (ii + wv) + ii * wv
                out_v[pl.ds(e0, L)] = acc + bias
                return carry

            lax.fori_loop(0, CHUNK // L, group, 0)

        pltpu.sync_copy(out_v, out_hbm.at[pl.ds(base, b_per_w)])

    return k(upidx, uoff, ipidx, ioff, upair, ipair, params)


def kernel(user_idx, item_idx, user_table, item_table, w, b):
    ui = user_idx.astype(jnp.int32)
    ii = item_idx.astype(jnp.int32)
    upair = user_table.reshape(-1, 2 * D)
    ipair = item_table.reshape(-1, 2 * D)
    params = jnp.concatenate(
        [jnp.repeat(w.astype(jnp.float32), L),
         jnp.broadcast_to(b.astype(jnp.float32), (L,))])
    return _fm_sc(ui >> 1, (ui & 1) << 6, ii >> 1, (ii & 1) << 6,
                  upair, ipair, params)
